# baseline (speedup 1.0000x reference)
import functools

import jax
import jax.numpy as jnp
from jax import lax
from jax.experimental import pallas as pl
from jax.experimental.pallas import tpu as pltpu
from jax.experimental.pallas import tpu_sc as plsc

MAXLEN = 200
VOCAB = 100000
CPLEN = 200
NTILE = 16
WCOL = 6272
WLAST = 5888
TAILOFF = (NTILE - 1) * WCOL + WLAST
TAILW = VOCAB - TAILOFF
SIDEW = 2 * TAILW


def _build_sc_gather():
    mesh = plsc.VectorSubcoreMesh(core_axis_name="c", subcore_axis_name="s")

    @functools.partial(
        pl.kernel,
        mesh=mesh,
        out_type=(
            jax.ShapeDtypeStruct((VOCAB,), jnp.float32),
            jax.ShapeDtypeStruct((VOCAB,), jnp.float32),
            jax.ShapeDtypeStruct((CPLEN,), jnp.float32),
        ),
        scratch_types=(
            pltpu.VMEM((16,), jnp.int32),
            pltpu.VMEM((8, WCOL), jnp.float32),
            pltpu.VMEM((CPLEN,), jnp.float32),
            pltpu.VMEM((SIDEW,), jnp.float32),
            pltpu.SemaphoreType.DMA,
            pltpu.SemaphoreType.DMA,
        ),
    )
    def gather_rows(rule_hbm, token_hbm, copy_hbm, side_hbm, len_hbm,
                    rule_out, token_out, copy_out,
                    len_v, buf, cbuf, sbuf, sem, sem2):
        cid = lax.axis_index("c")
        sid = lax.axis_index("s")

        pltpu.sync_copy(len_hbm.at[pl.ds(0, 16)], len_v)
        l0 = len_v[...][0]
        a = l0 - 1
        r = lax.rem(a, MAXLEN)
        l = jnp.where(r < 0, r + MAXLEN, r)

        lb = pl.multiple_of((l // 8) * 8, 8)
        lmod = l - lb

        for k, (tab, out) in enumerate(
                ((rule_hbm, rule_out), (token_hbm, token_out))):
            @pl.when(cid == k)
            def _(tab=tab, out=out):
                def fetch(col, wa, wb):
                    ca = pltpu.async_copy(
                        tab.at[pl.ds(lb, 8), pl.ds(col, wa)],
                        buf.at[:, pl.ds(0, wa)], sem)
                    cb = pltpu.async_copy(
                        tab.at[pl.ds(lb, 8), pl.ds(col + wa, wb)],
                        buf.at[:, pl.ds(wa, wb)], sem2)
                    ca.wait()
                    pltpu.sync_copy(buf.at[lmod, pl.ds(0, wa)],
                                    out.at[pl.ds(col, wa)])
                    cb.wait()
                    pltpu.sync_copy(buf.at[lmod, pl.ds(wa, wb)],
                                    out.at[pl.ds(col + wa, wb)])

                @pl.when(sid < NTILE - 1)
                def _():
                    fetch(sid * WCOL, 3200, WCOL - 3200)

                @pl.when(sid == NTILE - 1)
                def _():
                    fetch((NTILE - 1) * WCOL, 2944, WLAST - 2944)

        @pl.when((sid == 0) & (cid == 0))
        def _():
            pltpu.async_copy(copy_hbm.at[l], cbuf, sem).wait()
            pltpu.sync_copy(cbuf, copy_out)

        @pl.when((sid == 0) & (cid == 1))
        def _():
            pltpu.async_copy(side_hbm.at[l], sbuf, sem).wait()
            pltpu.sync_copy(sbuf.at[pl.ds(0, TAILW)],
                            rule_out.at[pl.ds(TAILOFF, TAILW)])
            pltpu.sync_copy(sbuf.at[pl.ds(TAILW, TAILW)],
                            token_out.at[pl.ds(TAILOFF, TAILW)])

    return gather_rows


_sc_gather = _build_sc_gather()


def kernel(rule_prob, token_prob, copy_prob, length):
    side = jnp.concatenate(
        (jax.lax.slice(rule_prob, (0, TAILOFF), (MAXLEN, VOCAB)),
         jax.lax.slice(token_prob, (0, TAILOFF), (MAXLEN, VOCAB))),
        axis=1)
    r, t, c = _sc_gather(rule_prob, token_prob, copy_prob, side, length)
    return (r, t, c, length)


# speedup vs baseline: 1.0351x; 1.0351x over previous
import functools

import jax
import jax.numpy as jnp
from jax import lax
from jax.experimental import pallas as pl
from jax.experimental.pallas import tpu as pltpu
from jax.experimental.pallas import tpu_sc as plsc

MAXLEN = 200
VOCAB = 100000
CPLEN = 200
NTILE = 16
WCOL = 6272
WLAST = 5888
TAILOFF = (NTILE - 1) * WCOL + WLAST
TAILW = VOCAB - TAILOFF
SIDEW = 2 * TAILW


def _build_sc_gather():
    mesh = plsc.VectorSubcoreMesh(core_axis_name="c", subcore_axis_name="s")

    @functools.partial(
        pl.kernel,
        mesh=mesh,
        out_type=(
            jax.ShapeDtypeStruct((VOCAB,), jnp.float32),
            jax.ShapeDtypeStruct((VOCAB,), jnp.float32),
            jax.ShapeDtypeStruct((CPLEN,), jnp.float32),
            jax.ShapeDtypeStruct((1024,), jnp.int32),
        ),
        scratch_types=(
            pltpu.VMEM((16,), jnp.int32),
            pltpu.VMEM((8, WCOL), jnp.float32),
            pltpu.VMEM((CPLEN,), jnp.float32),
            pltpu.VMEM((TAILW,), jnp.float32),
            pltpu.VMEM((1024,), jnp.int32),
            pltpu.SemaphoreType.DMA,
        ),
    )
    def gather_rows(rule_hbm, token_hbm, copy_hbm, side_hbm, len_hbm,
                    rule_out, token_out, copy_out, len_out,
                    len_v, buf, cbuf, sbuf, lbuf, sem):
        cid = lax.axis_index("c")
        sid = lax.axis_index("s")

        pltpu.sync_copy(len_hbm.at[pl.ds(0, 16)], len_v)
        l0 = len_v[...][0]
        a = l0 - 1
        r = lax.rem(a, MAXLEN)
        l = jnp.where(r < 0, r + MAXLEN, r)

        lb = pl.multiple_of((l // 8) * 8, 8)
        lmod = l - lb

        for k, (tab, out) in enumerate(
                ((rule_hbm, rule_out), (token_hbm, token_out))):
            @pl.when(cid == k)
            def _(tab=tab, out=out):
                @pl.when(sid < NTILE - 1)
                def _():
                    col = sid * WCOL
                    pltpu.async_copy(
                        tab.at[pl.ds(lb, 8), pl.ds(col, WCOL)], buf, sem
                    ).wait()
                    pltpu.sync_copy(buf.at[lmod],
                                    out.at[pl.ds(col, WCOL)])

                @pl.when(sid == NTILE - 1)
                def _():
                    col = (NTILE - 1) * WCOL
                    bl = buf.at[:, pl.ds(0, WLAST)]
                    pltpu.async_copy(
                        tab.at[pl.ds(lb, 8), pl.ds(col, WLAST)], bl, sem
                    ).wait()
                    pltpu.sync_copy(buf.at[lmod, pl.ds(0, WLAST)],
                                    out.at[pl.ds(col, WLAST)])

        @pl.when((sid == 0) & (cid == 0))
        def _():
            pltpu.async_copy(copy_hbm.at[l], cbuf, sem).wait()
            pltpu.sync_copy(cbuf, copy_out)

        @pl.when((sid == 0) & (cid == 1))
        def _():
            pltpu.async_copy(side_hbm.at[l], sbuf, sem).wait()
            pltpu.sync_copy(sbuf, rule_out.at[pl.ds(TAILOFF, TAILW)])

        @pl.when((sid == 1) & (cid == 1))
        def _():
            pltpu.async_copy(side_hbm.at[MAXLEN + l], sbuf, sem).wait()
            pltpu.sync_copy(sbuf, token_out.at[pl.ds(TAILOFF, TAILW)])

        @pl.when((sid == 1) & (cid == 0))
        def _():
            pltpu.async_copy(len_hbm, lbuf, sem).wait()
            pltpu.sync_copy(lbuf, len_out)

    return gather_rows


_sc_gather = _build_sc_gather()


def kernel(rule_prob, token_prob, copy_prob, length):
    side = jnp.concatenate(
        (jax.lax.slice(rule_prob, (0, TAILOFF), (MAXLEN, VOCAB)),
         jax.lax.slice(token_prob, (0, TAILOFF), (MAXLEN, VOCAB))),
        axis=0)
    r, t, c, ln = _sc_gather(rule_prob, token_prob, copy_prob, side, length)
    return (r, t, c, ln)
